# baseline (speedup 1.0000x reference)
import functools
import math

import jax
import jax.numpy as jnp
from jax import lax
from jax.experimental import pallas as pl
from jax.experimental.pallas import tpu as pltpu
from jax.experimental.pallas import tpu_sc as plsc

N = 10000
E = 160000
D = 256
H = 8
DH = 32
NC = 2
NS = 16
QC = 4
WQ_ = D // QC
HPQ = H // QC
INV_SQRT_DH = 1.0 / math.sqrt(DH)

BN = 2000
BE = 2000

CH = 128
NCH = E // CH
TRIPS = (NCH + NS - 1) // NS
NP = 10240
NPT = NP // NS

F32 = jnp.float32


def _qkv_body(h_ref, wq_ref, bq_ref, wk_ref, bk_ref, wv_ref, bv_ref,
              q_ref, k_ref, v_ref):
    hb = h_ref[...].astype(jnp.bfloat16)
    for w_ref, b_ref, o_ref in ((wq_ref, bq_ref, q_ref),
                                (wk_ref, bk_ref, k_ref),
                                (wv_ref, bv_ref, v_ref)):
        r = jnp.dot(hb, w_ref[...].astype(jnp.bfloat16),
                    preferred_element_type=F32) + b_ref[...]
        for q in range(QC):
            o_ref[q] = r[:, q * WQ_:(q + 1) * WQ_]


def _project_qkv(h, WQ, bQ, WK, bK, WV, bV):
    wspec = pl.BlockSpec((D, D), lambda i: (0, 0))
    bspec = pl.BlockSpec((1, D), lambda i: (0, 0))
    ospec = pl.BlockSpec((QC, BN, WQ_), lambda i: (0, i, 0))
    oshape = jax.ShapeDtypeStruct((QC, N, WQ_), F32)
    return pl.pallas_call(
        _qkv_body,
        grid=(N // BN,),
        in_specs=[pl.BlockSpec((BN, D), lambda i: (i, 0)),
                  wspec, bspec, wspec, bspec, wspec, bspec],
        out_specs=[ospec, ospec, ospec],
        out_shape=[oshape, oshape, oshape],
    )(h, WQ, bQ.reshape(1, D), WK, bK.reshape(1, D), WV, bV.reshape(1, D))


def _pe_body(e_ref, w_ref, b_ref, o_ref):
    r = jnp.dot(e_ref[...].astype(jnp.bfloat16),
                w_ref[...].astype(jnp.bfloat16),
                preferred_element_type=F32) + b_ref[...]
    for q in range(QC):
        o_ref[q] = r[:, q * WQ_:(q + 1) * WQ_]


def _project_pe(e, Wpe, bpe):
    return pl.pallas_call(
        _pe_body,
        grid=(E // BE,),
        in_specs=[pl.BlockSpec((BE, D), lambda i: (i, 0)),
                  pl.BlockSpec((D, D), lambda i: (0, 0)),
                  pl.BlockSpec((1, D), lambda i: (0, 0))],
        out_specs=pl.BlockSpec((QC, BE, WQ_), lambda i: (0, i, 0)),
        out_shape=jax.ShapeDtypeStruct((QC, E, WQ_), F32),
    )(e, Wpe, bpe.reshape(1, D))


def _node_body(h_ref, wv1_ref, z1_ref, wv2_ref, z2_ref,
               woh_ref, boh_ref, w1_ref, b1_ref, w2_ref, b2_ref, out_ref):
    eps = 1e-6
    colh = lax.broadcasted_iota(jnp.int32, (16, WQ_), 1) // DH
    rowh = lax.broadcasted_iota(jnp.int32, (16, WQ_), 0)
    P = jnp.where(colh == rowh, 1.0, 0.0).astype(F32)
    hh = boh_ref[...]
    for q in range(QC):
        z1e = jnp.dot(1.0 / (z1_ref[q] + eps), P, preferred_element_type=F32)
        z2e = jnp.dot(1.0 / (z2_ref[q] + eps), P, preferred_element_type=F32)
        h_out_q = wv1_ref[q] * z1e + wv2_ref[q] * z2e
        hh = hh + jnp.dot(h_out_q.astype(jnp.bfloat16),
                          woh_ref[q * WQ_:(q + 1) * WQ_, :].astype(jnp.bfloat16),
                          preferred_element_type=F32)
    hh = h_ref[...] + hh
    y = jnp.maximum(jnp.dot(hh.astype(jnp.bfloat16),
                            w1_ref[...].astype(jnp.bfloat16),
                            preferred_element_type=F32)
                    + b1_ref[...], 0.0)
    out_ref[...] = hh + jnp.dot(y.astype(jnp.bfloat16),
                                w2_ref[...].astype(jnp.bfloat16),
                                preferred_element_type=F32) + b2_ref[...]


def _node_finish(h, wv1, z1, wv2, z2, WOh, bOh, W1h, b1h, W2h, b2h):
    return pl.pallas_call(
        _node_body,
        grid=(N // BN,),
        in_specs=[pl.BlockSpec((BN, D), lambda i: (i, 0)),
                  pl.BlockSpec((QC, BN, WQ_), lambda i: (0, i, 0)),
                  pl.BlockSpec((QC, BN, 16), lambda i: (0, i, 0)),
                  pl.BlockSpec((QC, BN, WQ_), lambda i: (0, i, 0)),
                  pl.BlockSpec((QC, BN, 16), lambda i: (0, i, 0)),
                  pl.BlockSpec((D, D), lambda i: (0, 0)),
                  pl.BlockSpec((1, D), lambda i: (0, 0)),
                  pl.BlockSpec((D, 2 * D), lambda i: (0, 0)),
                  pl.BlockSpec((1, 2 * D), lambda i: (0, 0)),
                  pl.BlockSpec((2 * D, D), lambda i: (0, 0)),
                  pl.BlockSpec((1, D), lambda i: (0, 0))],
        out_specs=pl.BlockSpec((BN, D), lambda i: (i, 0)),
        out_shape=jax.ShapeDtypeStruct((N, D), F32),
    )(h, wv1, z1, wv2, z2, WOh, bOh.reshape(1, D), W1h, b1h.reshape(1, 2 * D),
      W2h, b2h.reshape(1, D))


def _edge_body(e_ref, eo_ref, woe_ref, boe_ref, w1_ref, b1_ref, w2_ref, b2_ref,
               out_ref):
    ee = boe_ref[...]
    for q in range(QC):
        ee = ee + jnp.dot(eo_ref[q].astype(jnp.bfloat16),
                          woe_ref[q * WQ_:(q + 1) * WQ_, :].astype(jnp.bfloat16),
                          preferred_element_type=F32)
    ee = e_ref[...] + ee
    y = jnp.maximum(jnp.dot(ee.astype(jnp.bfloat16),
                            w1_ref[...].astype(jnp.bfloat16),
                            preferred_element_type=F32)
                    + b1_ref[...], 0.0)
    out_ref[...] = ee + jnp.dot(y.astype(jnp.bfloat16),
                                w2_ref[...].astype(jnp.bfloat16),
                                preferred_element_type=F32) + b2_ref[...]


def _edge_finish(e, eout, WOe, bOe, W1e, b1e, W2e, b2e):
    return pl.pallas_call(
        _edge_body,
        grid=(E // BE,),
        in_specs=[pl.BlockSpec((BE, D), lambda i: (i, 0)),
                  pl.BlockSpec((QC, BE, WQ_), lambda i: (0, i, 0)),
                  pl.BlockSpec((D, D), lambda i: (0, 0)),
                  pl.BlockSpec((1, D), lambda i: (0, 0)),
                  pl.BlockSpec((D, 2 * D), lambda i: (0, 0)),
                  pl.BlockSpec((1, 2 * D), lambda i: (0, 0)),
                  pl.BlockSpec((2 * D, D), lambda i: (0, 0)),
                  pl.BlockSpec((1, D), lambda i: (0, 0))],
        out_specs=pl.BlockSpec((BE, D), lambda i: (i, 0)),
        out_shape=jax.ShapeDtypeStruct((E, D), F32),
    )(e, eout, WOe, bOe.reshape(1, D), W1e, b1e.reshape(1, 2 * D), W2e,
      b2e.reshape(1, D))


def _sc_edge_body(full, ktab, qtab, vtab, petab, src, dst, rel, adj, *rest):
    if full:
        (wvo, zo,
         idx_s0, idx_d0, idx_d20, kb0, qb0, pb0, rb0, ab0, vb0, sb0,
         idx_s1, idx_d1, idx_d21, kb1, qb1, pb1, rb1, ab1, vb1, sb1,
         zbufz, wv_acc, z_acc,
         semk0, semq0, semp0, scv0, scz0, sce0, semis0, semid0, semr0, sema0,
         semk1, semq1, semp1, scv1, scz1, sce1, semis1, semid1, semr1, sema1,
         semv) = rest
        eout = None
    else:
        (eout, wvo, zo,
         idx_s0, idx_d0, idx_d20, kb0, qb0, pb0, rb0, ab0, vb0, sb0,
         idx_s1, idx_d1, idx_d21, kb1, qb1, pb1, rb1, ab1, vb1, sb1,
         zbufz, wv_acc, z_acc,
         semk0, semq0, semp0, scv0, scz0, sce0, semis0, semid0, semr0, sema0,
         semk1, semq1, semp1, scv1, scz1, sce1, semis1, semid1, semr1, sema1,
         semv) = rest
    c = lax.axis_index("c")
    s = lax.axis_index("s")
    zero16 = jnp.zeros((16,), F32)

    bufs = ((idx_s0, idx_d0, idx_d20, kb0, qb0, pb0, rb0, ab0, vb0, sb0,
             semk0, semq0, semp0, scv0, scz0, sce0, semis0, semid0, semr0,
             sema0),
            (idx_s1, idx_d1, idx_d21, kb1, qb1, pb1, rb1, ab1, vb1, sb1,
             semk1, semq1, semp1, scv1, scz1, sce1, semis1, semid1, semr1,
             sema1))

    def _zero_sb(r, _):
        sb0[r, :] = zero16
        sb1[r, :] = zero16
        zbufz[r, :] = zero16
        return 0
    lax.fori_loop(0, CH, _zero_sb, 0)

    def _zero_kb(r, _):
        for j in range(WQ_ // 16):
            kb0[r, pl.ds(16 * j, 16)] = zero16
        return 0

    def _zero_accs():
        lax.fori_loop(0, CH, _zero_kb, 0)
        for t in range(NPT // CH):
            pltpu.sync_copy(kb0, wv_acc.at[pl.ds(s * NPT + t * CH, CH)])
            pltpu.sync_copy(zbufz, z_acc.at[pl.ds(s * NPT + t * CH, CH)])

    _zero_accs()
    plsc.subcore_barrier()

    iota16 = lax.iota(jnp.int32, 16)

    def _run_subphase(p):
        qq = c * 2 + p
        srcr = src
        dstr = dst

        def prefetch_idx(i, bs, drain):
            (idx_s, idx_d, idx_d2, kb, qb, pb, rb, ab, vb, sb,
             semk, semq, semp, scv, scz, sce, semis, semid, semr,
             sema) = bs
            cid = s + i * NS

            @pl.when(cid < NCH)
            def _():
                base = cid * CH

                def _drain():
                    pltpu.make_async_copy(vb, wv_acc.at[idx_d], scv).wait()
                    pltpu.make_async_copy(sb, z_acc.at[idx_d], scz).wait()
                    if not full:
                        pltpu.make_async_copy(
                            kb, eout.at[pl.ds(qq * E + base, CH)], sce).wait()

                if drain == "always":
                    _drain()
                elif drain == "dynamic":
                    pl.when(i >= 2)(_drain)
                pltpu.async_copy(srcr.at[pl.ds(base, CH)], idx_s, semis)
                pltpu.async_copy(dstr.at[pl.ds(base, CH)], idx_d, semid)
                if not full:
                    pltpu.async_copy(petab.at[pl.ds(qq * E + base, CH)], pb,
                                     semp)
                else:
                    pltpu.async_copy(rel.at[pl.ds(base, CH)], rb, semr)
                    pltpu.async_copy(adj.at[pl.ds(base, CH)], ab, sema)

        def launch_gathers(i, bs):
            (idx_s, idx_d, idx_d2, kb, qb, pb, rb, ab, vb, sb,
             semk, semq, semp, scv, scz, sce, semis, semid, semr,
             sema) = bs
            cid = s + i * NS

            @pl.when(cid < NCH)
            def _():
                base = cid * CH
                pltpu.make_async_copy(srcr.at[pl.ds(base, CH)], idx_s,
                                      semis).wait()
                pltpu.make_async_copy(dstr.at[pl.ds(base, CH)], idx_d,
                                      semid).wait()
                off = qq * N
                for j in range(CH // 16):
                    sl = pl.ds(16 * j, 16)
                    idx_s[sl] = idx_s[sl] + off
                    idx_d2[sl] = idx_d[sl] + off
                pltpu.async_copy(ktab.at[idx_s], kb, semk)
                pltpu.async_copy(qtab.at[idx_d2], qb, semq)

        def compute(i, bs, bs_next):
            (idx_s, idx_d, idx_d2, kb, qb, pb, rb, ab, vb, sb,
             semk, semq, semp, scv, scz, sce, semis, semid, semr,
             sema) = bs
            cid = s + i * NS

            @pl.when(cid < NCH)
            def _():
                base = cid * CH
                cv = pltpu.async_copy(vtab.at[idx_s], vb, semv)
                pltpu.make_async_copy(ktab.at[idx_s], kb, semk).wait()
                pltpu.make_async_copy(qtab.at[idx_d2], qb, semq).wait()
                if not full:
                    pltpu.make_async_copy(
                        petab.at[pl.ds(qq * E + base, CH)], pb, semp).wait()
                else:
                    pltpu.make_async_copy(rel.at[pl.ds(base, CH)], rb,
                                          semr).wait()
                    pltpu.make_async_copy(adj.at[pl.ds(base, CH)], ab,
                                          sema).wait()

                def rowA(r, _):
                    for j in range(WQ_ // 16):
                        sl = pl.ds(16 * j, 16)
                        if full:
                            kb[r, sl] = kb[r, sl] * qb[r, sl] * INV_SQRT_DH
                        else:
                            kb[r, sl] = kb[r, sl] * (qb[r, sl]
                                                     * (pb[r, sl] * INV_SQRT_DH))
                    return 0
                lax.fori_loop(0, CH, rowA, 0, unroll=2)

                launch_gathers(i + 1, bs_next)

                if not full:
                    pltpu.async_copy(kb, eout.at[pl.ds(qq * E + base, CH)],
                                     sce)

                def grp(g, _):
                    rows = g * 16 + iota16
                    for hh in range(HPQ):
                        acc = zero16
                        colv = jnp.zeros((16,), jnp.int32) + (hh * DH)
                        for _d in range(DH):
                            acc = acc + plsc.load_gather(kb, [rows, colv])
                            colv = colv + 1
                        if full:
                            relcol = (jnp.zeros((16,), jnp.int32)
                                      + (qq * HPQ + hh))
                            relv = plsc.load_gather(rb, [rows, relcol])
                            sv = jnp.exp(jnp.clip(acc + relv, -5.0, 5.0)
                                         * ab[pl.ds(g * 16, 16)])
                        else:
                            sv = jnp.exp(jnp.clip(acc, -5.0, 5.0))
                        plsc.store_scatter(
                            sb, [rows, jnp.zeros((16,), jnp.int32) + hh], sv)
                    return 0
                lax.fori_loop(0, CH // 16, grp, 0)

                cv.wait()

                def rowC(r, _):
                    srow = sb[r, :]
                    for j in range(WQ_ // 16):
                        sl = pl.ds(16 * j, 16)
                        vb[r, sl] = vb[r, sl] * srow[j // 2]
                    return 0
                lax.fori_loop(0, CH, rowC, 0)

                pltpu.async_copy(vb, wv_acc.at[idx_d], scv, add=True)
                pltpu.async_copy(sb, z_acc.at[idx_d], scz, add=True)

        prefetch_idx(0, bufs[0], "never")
        launch_gathers(0, bufs[0])

        def chunk_pair(ii, _):
            i = ii * 2
            prefetch_idx(i + 1, bufs[1], "dynamic")
            compute(i, bufs[0], bufs[1])
            prefetch_idx(i + 2, bufs[0], "always")
            compute(i + 1, bufs[1], bufs[0])
            return 0

        lax.fori_loop(0, TRIPS // 2, chunk_pair, 0)
        if TRIPS % 2:
            compute(TRIPS - 1, bufs[(TRIPS - 1) % 2],
                    bufs[TRIPS % 2])

        for bs in bufs:
            (idx_s, idx_d, idx_d2, kb, qb, pb, rb, ab, vb, sb,
             semk, semq, semp, scv, scz, sce, semis, semid, semr,
             sema) = bs
            pltpu.make_async_copy(vb, wv_acc.at[idx_d], scv).wait()
            pltpu.make_async_copy(sb, z_acc.at[idx_d], scz).wait()
            if not full:
                pltpu.make_async_copy(
                    kb, eout.at[pl.ds(0, CH)], sce).wait()

        plsc.subcore_barrier()
        for t in range(NPT // CH):
            pltpu.sync_copy(wv_acc.at[pl.ds(s * NPT + t * CH, CH)],
                            wvo.at[pl.ds(qq * NP + s * NPT + t * CH, CH)])
            pltpu.sync_copy(z_acc.at[pl.ds(s * NPT + t * CH, CH)],
                            zo.at[pl.ds(qq * NP + s * NPT + t * CH, CH)])
        _zero_accs()
        plsc.subcore_barrier()

    _run_subphase(0)
    _run_subphase(1)


def _sc_scratch():
    bufset = [
        pltpu.VMEM((CH,), jnp.int32),
        pltpu.VMEM((CH,), jnp.int32),
        pltpu.VMEM((CH,), jnp.int32),
        pltpu.VMEM((CH, WQ_), F32),
        pltpu.VMEM((CH, WQ_), F32),
        pltpu.VMEM((CH, WQ_), F32),
        pltpu.VMEM((CH, H), F32),
        pltpu.VMEM((CH,), F32),
        pltpu.VMEM((CH, WQ_), F32),
        pltpu.VMEM((CH, 16), F32),
    ]
    return bufset + bufset + [
        pltpu.VMEM((CH, 16), F32),
        pltpu.VMEM_SHARED((NP, WQ_), F32),
        pltpu.VMEM_SHARED((NP, 16), F32),
    ] + [pltpu.SemaphoreType.DMA] * 21


_SC_MESH = plsc.VectorSubcoreMesh(core_axis_name="c", subcore_axis_name="s",
                                  num_cores=NC, num_subcores=NS)
_SC_PARAMS = pltpu.CompilerParams(needs_layout_passes=False,
                                  use_tc_tiling_on_sc=False)


def _edge_stage_sc1(qtab, ktab, vtab, petab, src, dst, rel, adj):
    out_type = (jax.ShapeDtypeStruct((QC * E, WQ_), F32),
                jax.ShapeDtypeStruct((QC * NP, WQ_), F32),
                jax.ShapeDtypeStruct((QC * NP, 16), F32))
    run = pl.kernel(functools.partial(_sc_edge_body, False),
                    out_type=out_type, mesh=_SC_MESH,
                    scratch_types=_sc_scratch(),
                    compiler_params=_SC_PARAMS)
    eout, wv1, z1 = run(ktab, qtab, vtab, petab, src, dst, rel, adj)
    return (eout.reshape(QC, E, WQ_), wv1.reshape(QC, NP, WQ_),
            z1.reshape(QC, NP, 16))


def _edge_stage_sc2(qtab, ktab, vtab, petab, src2, dst2, rel, adj):
    out_type = (jax.ShapeDtypeStruct((QC * NP, WQ_), F32),
                jax.ShapeDtypeStruct((QC * NP, 16), F32))
    run = pl.kernel(functools.partial(_sc_edge_body, True),
                    out_type=out_type, mesh=_SC_MESH,
                    scratch_types=_sc_scratch(),
                    compiler_params=_SC_PARAMS)
    wv2, z2 = run(ktab, qtab, vtab, petab, src2, dst2, rel, adj)
    return (wv2.reshape(QC, NP, WQ_), z2.reshape(QC, NP, 16))


def kernel(h, e, edge_index, full_edge_index, adj2, rel_pos_3d,
           WQ, bQ, WK, bK, WV, bV, Wpe, bpe, WOh, bOh, WOe, bOe,
           W1h, b1h, W2h, b2h, W1e, b1e, W2e, b2e):
    qtab, ktab, vtab = _project_qkv(h, WQ, bQ, WK, bK, WV, bV)
    petab = _project_pe(e, Wpe, bpe)
    ktab = ktab.reshape(QC * N, WQ_)
    qtab = qtab.reshape(QC * N, WQ_)
    vtab = vtab.reshape(QC * N, WQ_)
    petab = petab.reshape(QC * E, WQ_)
    src, dst = edge_index[0], edge_index[1]
    src2, dst2 = full_edge_index[0], full_edge_index[1]
    eout, wv1, z1 = _edge_stage_sc1(
        qtab, ktab, vtab, petab, src, dst, rel_pos_3d, adj2)
    wv2, z2 = _edge_stage_sc2(
        qtab, ktab, vtab, petab, src2, dst2, rel_pos_3d, adj2)
    ee = _edge_finish(e, eout, WOe, bOe, W1e, b1e, W2e, b2e)
    hh = _node_finish(h, wv1, z1, wv2, z2, WOh, bOh, W1h, b1h, W2h, b2h)
    return (hh, ee)


# speedup vs baseline: 1.2447x; 1.2447x over previous
import functools
import math

import jax
import jax.numpy as jnp
from jax import lax
from jax.experimental import pallas as pl
from jax.experimental.pallas import tpu as pltpu
from jax.experimental.pallas import tpu_sc as plsc

N = 10000
E = 160000
D = 256
H = 8
DH = 32
NC = 2
NS = 16
QC = 4
WQ_ = D // QC
HPQ = H // QC
INV_SQRT_DH = 1.0 / math.sqrt(DH)

BN = 2000
BE = 2000

CH = 128
NCH = E // CH
TRIPS = (NCH + NS - 1) // NS
NP = 10240
NPT = NP // NS

F32 = jnp.float32


def _qkv_body(h_ref, wq_ref, bq_ref, wk_ref, bk_ref, wv_ref, bv_ref,
              q_ref, k_ref, v_ref):
    hb = h_ref[...].astype(jnp.bfloat16)
    for w_ref, b_ref, o_ref in ((wq_ref, bq_ref, q_ref),
                                (wk_ref, bk_ref, k_ref),
                                (wv_ref, bv_ref, v_ref)):
        r = jnp.dot(hb, w_ref[...].astype(jnp.bfloat16),
                    preferred_element_type=F32) + b_ref[...]
        for q in range(QC):
            o_ref[q] = r[:, q * WQ_:(q + 1) * WQ_]


def _project_qkv(h, WQ, bQ, WK, bK, WV, bV):
    wspec = pl.BlockSpec((D, D), lambda i: (0, 0))
    bspec = pl.BlockSpec((1, D), lambda i: (0, 0))
    ospec = pl.BlockSpec((QC, BN, WQ_), lambda i: (0, i, 0))
    oshape = jax.ShapeDtypeStruct((QC, N, WQ_), F32)
    return pl.pallas_call(
        _qkv_body,
        grid=(N // BN,),
        in_specs=[pl.BlockSpec((BN, D), lambda i: (i, 0)),
                  wspec, bspec, wspec, bspec, wspec, bspec],
        out_specs=[ospec, ospec, ospec],
        out_shape=[oshape, oshape, oshape],
    )(h, WQ, bQ.reshape(1, D), WK, bK.reshape(1, D), WV, bV.reshape(1, D))


def _pe_body(e_ref, w_ref, b_ref, o_ref):
    r = jnp.dot(e_ref[...].astype(jnp.bfloat16),
                w_ref[...].astype(jnp.bfloat16),
                preferred_element_type=F32) + b_ref[...]
    for q in range(QC):
        o_ref[q] = r[:, q * WQ_:(q + 1) * WQ_]


def _project_pe(e, Wpe, bpe):
    return pl.pallas_call(
        _pe_body,
        grid=(E // BE,),
        in_specs=[pl.BlockSpec((BE, D), lambda i: (i, 0)),
                  pl.BlockSpec((D, D), lambda i: (0, 0)),
                  pl.BlockSpec((1, D), lambda i: (0, 0))],
        out_specs=pl.BlockSpec((QC, BE, WQ_), lambda i: (0, i, 0)),
        out_shape=jax.ShapeDtypeStruct((QC, E, WQ_), F32),
    )(e, Wpe, bpe.reshape(1, D))


def _node_body(h_ref, wv1_ref, z1_ref, wv2_ref, z2_ref,
               woh_ref, boh_ref, w1_ref, b1_ref, w2_ref, b2_ref, out_ref):
    eps = 1e-6
    colh = lax.broadcasted_iota(jnp.int32, (16, WQ_), 1) // DH
    rowh = lax.broadcasted_iota(jnp.int32, (16, WQ_), 0)
    P = jnp.where(colh == rowh, 1.0, 0.0).astype(F32)
    hh = boh_ref[...]
    for q in range(QC):
        z1e = jnp.dot(1.0 / (z1_ref[q] + eps), P, preferred_element_type=F32)
        z2e = jnp.dot(1.0 / (z2_ref[q] + eps), P, preferred_element_type=F32)
        h_out_q = wv1_ref[q] * z1e + wv2_ref[q] * z2e
        hh = hh + jnp.dot(h_out_q.astype(jnp.bfloat16),
                          woh_ref[q * WQ_:(q + 1) * WQ_, :].astype(jnp.bfloat16),
                          preferred_element_type=F32)
    hh = h_ref[...] + hh
    y = jnp.maximum(jnp.dot(hh.astype(jnp.bfloat16),
                            w1_ref[...].astype(jnp.bfloat16),
                            preferred_element_type=F32)
                    + b1_ref[...], 0.0)
    out_ref[...] = hh + jnp.dot(y.astype(jnp.bfloat16),
                                w2_ref[...].astype(jnp.bfloat16),
                                preferred_element_type=F32) + b2_ref[...]


def _node_finish(h, wv1, z1, wv2, z2, WOh, bOh, W1h, b1h, W2h, b2h):
    return pl.pallas_call(
        _node_body,
        grid=(N // BN,),
        in_specs=[pl.BlockSpec((BN, D), lambda i: (i, 0)),
                  pl.BlockSpec((QC, BN, WQ_), lambda i: (0, i, 0)),
                  pl.BlockSpec((QC, BN, 16), lambda i: (0, i, 0)),
                  pl.BlockSpec((QC, BN, WQ_), lambda i: (0, i, 0)),
                  pl.BlockSpec((QC, BN, 16), lambda i: (0, i, 0)),
                  pl.BlockSpec((D, D), lambda i: (0, 0)),
                  pl.BlockSpec((1, D), lambda i: (0, 0)),
                  pl.BlockSpec((D, 2 * D), lambda i: (0, 0)),
                  pl.BlockSpec((1, 2 * D), lambda i: (0, 0)),
                  pl.BlockSpec((2 * D, D), lambda i: (0, 0)),
                  pl.BlockSpec((1, D), lambda i: (0, 0))],
        out_specs=pl.BlockSpec((BN, D), lambda i: (i, 0)),
        out_shape=jax.ShapeDtypeStruct((N, D), F32),
    )(h, wv1, z1, wv2, z2, WOh, bOh.reshape(1, D), W1h, b1h.reshape(1, 2 * D),
      W2h, b2h.reshape(1, D))


def _edge_body(e_ref, eo_ref, woe_ref, boe_ref, w1_ref, b1_ref, w2_ref, b2_ref,
               out_ref):
    ee = boe_ref[...]
    for q in range(QC):
        ee = ee + jnp.dot(eo_ref[q].astype(jnp.bfloat16),
                          woe_ref[q * WQ_:(q + 1) * WQ_, :].astype(jnp.bfloat16),
                          preferred_element_type=F32)
    ee = e_ref[...] + ee
    y = jnp.maximum(jnp.dot(ee.astype(jnp.bfloat16),
                            w1_ref[...].astype(jnp.bfloat16),
                            preferred_element_type=F32)
                    + b1_ref[...], 0.0)
    out_ref[...] = ee + jnp.dot(y.astype(jnp.bfloat16),
                                w2_ref[...].astype(jnp.bfloat16),
                                preferred_element_type=F32) + b2_ref[...]


def _edge_finish(e, eout, WOe, bOe, W1e, b1e, W2e, b2e):
    return pl.pallas_call(
        _edge_body,
        grid=(E // BE,),
        in_specs=[pl.BlockSpec((BE, D), lambda i: (i, 0)),
                  pl.BlockSpec((QC, BE, WQ_), lambda i: (0, i, 0)),
                  pl.BlockSpec((D, D), lambda i: (0, 0)),
                  pl.BlockSpec((1, D), lambda i: (0, 0)),
                  pl.BlockSpec((D, 2 * D), lambda i: (0, 0)),
                  pl.BlockSpec((1, 2 * D), lambda i: (0, 0)),
                  pl.BlockSpec((2 * D, D), lambda i: (0, 0)),
                  pl.BlockSpec((1, D), lambda i: (0, 0))],
        out_specs=pl.BlockSpec((BE, D), lambda i: (i, 0)),
        out_shape=jax.ShapeDtypeStruct((E, D), F32),
    )(e, eout, WOe, bOe.reshape(1, D), W1e, b1e.reshape(1, 2 * D), W2e,
      b2e.reshape(1, D))


def _sc_edge_body(full, ktab, qtab, vtab, petab, src, dst, rel, adj, *rest):
    if full:
        (wvo, zo,
         idx_s0, idx_d0, idx_d20, kb0, qb0, pb0, rb0, ab0, vb0, sb0,
         idx_s1, idx_d1, idx_d21, kb1, qb1, pb1, rb1, ab1, vb1, sb1,
         zbufz, wv_acc, z_acc,
         semk0, semq0, semp0, scv0, scz0, sce0, semis0, semid0, semr0, sema0,
         semk1, semq1, semp1, scv1, scz1, sce1, semis1, semid1, semr1, sema1,
         semv) = rest
        eout = None
    else:
        (eout, wvo, zo,
         idx_s0, idx_d0, idx_d20, kb0, qb0, pb0, rb0, ab0, vb0, sb0,
         idx_s1, idx_d1, idx_d21, kb1, qb1, pb1, rb1, ab1, vb1, sb1,
         zbufz, wv_acc, z_acc,
         semk0, semq0, semp0, scv0, scz0, sce0, semis0, semid0, semr0, sema0,
         semk1, semq1, semp1, scv1, scz1, sce1, semis1, semid1, semr1, sema1,
         semv) = rest
    c = lax.axis_index("c")
    s = lax.axis_index("s")
    zero16 = jnp.zeros((16,), F32)

    bufs = ((idx_s0, idx_d0, idx_d20, kb0, qb0, pb0, rb0, ab0, vb0, sb0,
             semk0, semq0, semp0, scv0, scz0, sce0, semis0, semid0, semr0,
             sema0),
            (idx_s1, idx_d1, idx_d21, kb1, qb1, pb1, rb1, ab1, vb1, sb1,
             semk1, semq1, semp1, scv1, scz1, sce1, semis1, semid1, semr1,
             sema1))

    def _zero_sb(r, _):
        sb0[r, :] = zero16
        sb1[r, :] = zero16
        zbufz[r, :] = zero16
        return 0
    lax.fori_loop(0, CH, _zero_sb, 0)

    def _zero_kb(r, _):
        for j in range(WQ_ // 16):
            kb0[r, pl.ds(16 * j, 16)] = zero16
        return 0

    def _zero_accs():
        lax.fori_loop(0, CH, _zero_kb, 0)
        for t in range(NPT // CH):
            pltpu.sync_copy(kb0, wv_acc.at[pl.ds(s * NPT + t * CH, CH)])
            pltpu.sync_copy(zbufz, z_acc.at[pl.ds(s * NPT + t * CH, CH)])

    _zero_accs()
    plsc.subcore_barrier()

    iota16 = lax.iota(jnp.int32, 16)

    def _run_subphase(p):
        qq = c * 2 + p
        srcr = src
        dstr = dst

        def prefetch_idx(i, bs, drain):
            (idx_s, idx_d, idx_d2, kb, qb, pb, rb, ab, vb, sb,
             semk, semq, semp, scv, scz, sce, semis, semid, semr,
             sema) = bs
            cid = s + i * NS

            @pl.when(cid < NCH)
            def _():
                base = cid * CH

                def _drain():
                    pltpu.make_async_copy(vb, wv_acc.at[idx_d], scv).wait()
                    pltpu.make_async_copy(sb, z_acc.at[idx_d], scz).wait()
                    if not full:
                        pltpu.make_async_copy(
                            kb, eout.at[pl.ds(qq * E + base, CH)], sce).wait()

                if drain == "always":
                    _drain()
                elif drain == "dynamic":
                    pl.when(i >= 2)(_drain)
                pltpu.async_copy(srcr.at[pl.ds(base, CH)], idx_s, semis)
                pltpu.async_copy(dstr.at[pl.ds(base, CH)], idx_d, semid)
                if not full:
                    pltpu.async_copy(petab.at[pl.ds(qq * E + base, CH)], pb,
                                     semp)
                else:
                    pltpu.async_copy(rel.at[pl.ds(base, CH)], rb, semr)
                    pltpu.async_copy(adj.at[pl.ds(base, CH)], ab, sema)

        def launch_gathers(i, bs):
            (idx_s, idx_d, idx_d2, kb, qb, pb, rb, ab, vb, sb,
             semk, semq, semp, scv, scz, sce, semis, semid, semr,
             sema) = bs
            cid = s + i * NS

            @pl.when(cid < NCH)
            def _():
                base = cid * CH
                pltpu.make_async_copy(srcr.at[pl.ds(base, CH)], idx_s,
                                      semis).wait()
                pltpu.make_async_copy(dstr.at[pl.ds(base, CH)], idx_d,
                                      semid).wait()
                off = qq * N
                for j in range(CH // 16):
                    sl = pl.ds(16 * j, 16)
                    idx_s[sl] = idx_s[sl] + off
                    idx_d2[sl] = idx_d[sl] + off
                pltpu.async_copy(ktab.at[idx_s], kb, semk)
                pltpu.async_copy(qtab.at[idx_d2], qb, semq)

        def compute(i, bs, bs_next):
            (idx_s, idx_d, idx_d2, kb, qb, pb, rb, ab, vb, sb,
             semk, semq, semp, scv, scz, sce, semis, semid, semr,
             sema) = bs
            cid = s + i * NS

            @pl.when(cid < NCH)
            def _():
                base = cid * CH
                cv = pltpu.async_copy(vtab.at[idx_s], vb, semv)
                pltpu.make_async_copy(ktab.at[idx_s], kb, semk).wait()
                pltpu.make_async_copy(qtab.at[idx_d2], qb, semq).wait()
                if not full:
                    pltpu.make_async_copy(
                        petab.at[pl.ds(qq * E + base, CH)], pb, semp).wait()
                else:
                    pltpu.make_async_copy(rel.at[pl.ds(base, CH)], rb,
                                          semr).wait()
                    pltpu.make_async_copy(adj.at[pl.ds(base, CH)], ab,
                                          sema).wait()

                def rowA(r, _):
                    for j in range(WQ_ // 16):
                        sl = pl.ds(16 * j, 16)
                        if full:
                            kb[r, sl] = kb[r, sl] * qb[r, sl] * INV_SQRT_DH
                        else:
                            kb[r, sl] = kb[r, sl] * (qb[r, sl]
                                                     * (pb[r, sl] * INV_SQRT_DH))
                    return 0
                lax.fori_loop(0, CH, rowA, 0)

                launch_gathers(i + 1, bs_next)

                if not full:
                    pltpu.async_copy(kb, eout.at[pl.ds(qq * E + base, CH)],
                                     sce)

                def grp(g, _):
                    rows = g * 16 + iota16
                    for hh in range(HPQ):
                        acc = zero16
                        colv = jnp.zeros((16,), jnp.int32) + (hh * DH)
                        for _d in range(DH):
                            acc = acc + plsc.load_gather(kb, [rows, colv])
                            colv = colv + 1
                        if full:
                            relcol = (jnp.zeros((16,), jnp.int32)
                                      + (qq * HPQ + hh))
                            relv = plsc.load_gather(rb, [rows, relcol])
                            sv = jnp.exp(jnp.clip(acc + relv, -5.0, 5.0)
                                         * ab[pl.ds(g * 16, 16)])
                        else:
                            sv = jnp.exp(jnp.clip(acc, -5.0, 5.0))
                        plsc.store_scatter(
                            sb, [rows, jnp.zeros((16,), jnp.int32) + hh], sv)
                    return 0
                lax.fori_loop(0, CH // 16, grp, 0)

                cv.wait()

                def rowC(r, _):
                    srow = sb[r, :]
                    for j in range(WQ_ // 16):
                        sl = pl.ds(16 * j, 16)
                        vb[r, sl] = vb[r, sl] * srow[j // 2]
                    return 0
                lax.fori_loop(0, CH, rowC, 0)

                pltpu.async_copy(vb, wv_acc.at[idx_d], scv, add=True)
                pltpu.async_copy(sb, z_acc.at[idx_d], scz, add=True)

        prefetch_idx(0, bufs[0], "never")
        launch_gathers(0, bufs[0])

        def chunk_pair(ii, _):
            i = ii * 2
            prefetch_idx(i + 1, bufs[1], "dynamic")
            compute(i, bufs[0], bufs[1])
            prefetch_idx(i + 2, bufs[0], "always")
            compute(i + 1, bufs[1], bufs[0])
            return 0

        lax.fori_loop(0, TRIPS // 2, chunk_pair, 0)
        if TRIPS % 2:
            compute(TRIPS - 1, bufs[(TRIPS - 1) % 2],
                    bufs[TRIPS % 2])

        for bs in bufs:
            (idx_s, idx_d, idx_d2, kb, qb, pb, rb, ab, vb, sb,
             semk, semq, semp, scv, scz, sce, semis, semid, semr,
             sema) = bs
            pltpu.make_async_copy(vb, wv_acc.at[idx_d], scv).wait()
            pltpu.make_async_copy(sb, z_acc.at[idx_d], scz).wait()
            if not full:
                pltpu.make_async_copy(
                    kb, eout.at[pl.ds(0, CH)], sce).wait()

        plsc.subcore_barrier()
        for t in range(NPT // CH):
            pltpu.sync_copy(wv_acc.at[pl.ds(s * NPT + t * CH, CH)],
                            wvo.at[pl.ds(qq * NP + s * NPT + t * CH, CH)])
            pltpu.sync_copy(z_acc.at[pl.ds(s * NPT + t * CH, CH)],
                            zo.at[pl.ds(qq * NP + s * NPT + t * CH, CH)])
        _zero_accs()
        plsc.subcore_barrier()

    _run_subphase(0)
    _run_subphase(1)


def _sc_scratch():
    bufset = [
        pltpu.VMEM((CH,), jnp.int32),
        pltpu.VMEM((CH,), jnp.int32),
        pltpu.VMEM((CH,), jnp.int32),
        pltpu.VMEM((CH, WQ_), F32),
        pltpu.VMEM((CH, WQ_), F32),
        pltpu.VMEM((CH, WQ_), F32),
        pltpu.VMEM((CH, H), F32),
        pltpu.VMEM((CH,), F32),
        pltpu.VMEM((CH, WQ_), F32),
        pltpu.VMEM((CH, 16), F32),
    ]
    return bufset + bufset + [
        pltpu.VMEM((CH, 16), F32),
        pltpu.VMEM_SHARED((NP, WQ_), F32),
        pltpu.VMEM_SHARED((NP, 16), F32),
    ] + [pltpu.SemaphoreType.DMA] * 21


_SC_MESH = plsc.VectorSubcoreMesh(core_axis_name="c", subcore_axis_name="s",
                                  num_cores=NC, num_subcores=NS)
_SC_PARAMS = pltpu.CompilerParams(needs_layout_passes=False,
                                  use_tc_tiling_on_sc=False)


def _edge_stage_sc1(qtab, ktab, vtab, petab, src, dst, rel, adj):
    out_type = (jax.ShapeDtypeStruct((QC * E, WQ_), F32),
                jax.ShapeDtypeStruct((QC * NP, WQ_), F32),
                jax.ShapeDtypeStruct((QC * NP, 16), F32))
    run = pl.kernel(functools.partial(_sc_edge_body, False),
                    out_type=out_type, mesh=_SC_MESH,
                    scratch_types=_sc_scratch(),
                    compiler_params=_SC_PARAMS)
    eout, wv1, z1 = run(ktab, qtab, vtab, petab, src, dst, rel, adj)
    return (eout.reshape(QC, E, WQ_), wv1.reshape(QC, NP, WQ_),
            z1.reshape(QC, NP, 16))


def _edge_stage_sc2(qtab, ktab, vtab, petab, src2, dst2, rel, adj):
    out_type = (jax.ShapeDtypeStruct((QC * NP, WQ_), F32),
                jax.ShapeDtypeStruct((QC * NP, 16), F32))
    run = pl.kernel(functools.partial(_sc_edge_body, True),
                    out_type=out_type, mesh=_SC_MESH,
                    scratch_types=_sc_scratch(),
                    compiler_params=_SC_PARAMS)
    wv2, z2 = run(ktab, qtab, vtab, petab, src2, dst2, rel, adj)
    return (wv2.reshape(QC, NP, WQ_), z2.reshape(QC, NP, 16))


def kernel(h, e, edge_index, full_edge_index, adj2, rel_pos_3d,
           WQ, bQ, WK, bK, WV, bV, Wpe, bpe, WOh, bOh, WOe, bOe,
           W1h, b1h, W2h, b2h, W1e, b1e, W2e, b2e):
    qtab, ktab, vtab = _project_qkv(h, WQ, bQ, WK, bK, WV, bV)
    petab = _project_pe(e, Wpe, bpe)
    ktab = ktab.reshape(QC * N, WQ_)
    qtab = qtab.reshape(QC * N, WQ_)
    vtab = vtab.reshape(QC * N, WQ_)
    petab = petab.reshape(QC * E, WQ_)
    src, dst = edge_index[0], edge_index[1]
    src2, dst2 = full_edge_index[0], full_edge_index[1]
    eout, wv1, z1 = _edge_stage_sc1(
        qtab, ktab, vtab, petab, src, dst, rel_pos_3d, adj2)
    wv2, z2 = _edge_stage_sc2(
        qtab, ktab, vtab, petab, src2, dst2, rel_pos_3d, adj2)
    ee = _edge_finish(e, eout, WOe, bOe, W1e, b1e, W2e, b2e)
    hh = _node_finish(h, wv1, z1, wv2, z2, WOh, bOh, W1h, b1h, W2h, b2h)
    return (hh, ee)
